# Initial kernel scaffold; baseline (speedup 1.0000x reference)
import jax
import jax.numpy as jnp
from jax.experimental import pallas as pl


def kernel(x, segment_input_ids, seg_table):
    raise NotImplementedError("write your pallas kernel here")


# speedup vs baseline: 2.3596x; 2.3596x over previous
import jax
import jax.numpy as jnp
from jax.experimental import pallas as pl

_BS = 2048
_D = 1024


def _body(ids_ref, x_ref, tab_ref, o_ref):
    cond = ids_ref[0] == 0
    emb = jnp.where(cond, tab_ref[0, :], tab_ref[1, :])
    o_ref[...] = x_ref[...] + emb


def kernel(x, segment_input_ids, seg_table):
    B, S, D = x.shape
    N = B * S
    x2 = x.reshape(N, D)
    grid = N // _BS
    ids3 = segment_input_ids.astype(jnp.int32).reshape(grid, _BS, 1)
    out = pl.pallas_call(
        _body,
        grid=(grid,),
        in_specs=[
            pl.BlockSpec((1, _BS, 1), lambda i: (i, 0, 0)),
            pl.BlockSpec((_BS, D), lambda i: (i, 0)),
            pl.BlockSpec((2, D), lambda i: (0, 0)),
        ],
        out_specs=pl.BlockSpec((_BS, D), lambda i: (i, 0)),
        out_shape=jax.ShapeDtypeStruct((N, D), x.dtype),
    )(ids3, x2, seg_table)
    return out.reshape(B, S, D)
